# baseline (speedup 1.0000x reference)
import functools

import jax
import jax.numpy as jnp
from jax import lax
from jax.experimental import pallas as pl
from jax.experimental.pallas import tpu as pltpu
from jax.experimental.pallas import tpu_sc as plsc

HIDDEN = 768
NSLICE = HIDDEN // 16
EPS = 1e-5
INV_H = 1.0 / HIDDEN


def _lane_sum16(x):
    dnums = lax.GatherDimensionNumbers(
        offset_dims=(), collapsed_slice_dims=(0,), start_index_map=(0,))
    lane = lax.iota(jnp.int32, 16)
    for sh in (8, 4, 2, 1):
        perm = ((lane + sh) & 15).reshape(16, 1)
        x = x + lax.gather(x, perm, dnums, slice_sizes=(1,),
                           mode=lax.GatherScatterMode.PROMISE_IN_BOUNDS)
    return x


def _rsqrt16(x):
    i = plsc.bitcast(x, jnp.int32)
    i = jnp.int32(0x5F3759DF) - (i >> 1)
    y = plsc.bitcast(i, jnp.float32)
    for _ in range(3):
        y = y * (1.5 - 0.5 * x * y * y)
    return y


@functools.partial(jax.jit, static_argnames=("n_tokens",))
def _sc_embed_ln(iw, ip, it, word_emb, pos_emb, tok_type_emb, gamma, beta, *, n_tokens):
    info = plsc.get_sparse_core_info()
    nc, ns = info.num_cores, info.num_subcores
    nw = nc * ns
    t_per_w = n_tokens // nw
    C = 32
    n_chunks = t_per_w // C

    mesh = plsc.VectorSubcoreMesh(core_axis_name="c", subcore_axis_name="s")

    @functools.partial(
        pl.kernel,
        mesh=mesh,
        compiler_params=pltpu.CompilerParams(needs_layout_passes=False),
        out_type=jax.ShapeDtypeStruct((n_tokens, HIDDEN), jnp.float32),
        scratch_types=[
            pltpu.VMEM((t_per_w,), jnp.int32),
            pltpu.VMEM((t_per_w,), jnp.int32),
            pltpu.VMEM((t_per_w,), jnp.int32),
            pltpu.VMEM((2 * C, HIDDEN), jnp.float32),
            pltpu.VMEM((2 * C, HIDDEN), jnp.float32),
            pltpu.VMEM((2, HIDDEN), jnp.float32),
            pltpu.VMEM((HIDDEN,), jnp.float32),
            pltpu.VMEM((HIDDEN,), jnp.float32),
            pltpu.SMEM((C,), jnp.float32),
            pltpu.SMEM((C,), jnp.float32),
            pltpu.SMEM((C,), jnp.int32),
            pltpu.SemaphoreType.DMA,
            pltpu.SemaphoreType.DMA,
        ],
    )
    def k(iw_hbm, ip_hbm, it_hbm, wtab, ptab, ttab_hbm, g_hbm, b_hbm, out_hbm,
          idxw_all, idxp_all, idxt_all, wbig, pbig,
          ttab, gbuf, bbuf, mbuf, rbuf, tsm, gsem, ssem):
        wid = lax.axis_index("s") * nc + lax.axis_index("c")
        base = wid * t_per_w
        pltpu.sync_copy(iw_hbm.at[pl.ds(base, t_per_w)], idxw_all)
        pltpu.sync_copy(ip_hbm.at[pl.ds(base, t_per_w)], idxp_all)
        pltpu.sync_copy(it_hbm.at[pl.ds(base, t_per_w)], idxt_all)
        pltpu.sync_copy(ttab_hbm, ttab)
        pltpu.sync_copy(g_hbm, gbuf)
        pltpu.sync_copy(b_hbm, bbuf)

        def fire_gather(c):
            half = (c & 1) * C
            pltpu.async_copy(wtab.at[idxw_all.at[pl.ds(c * C, C)]],
                             wbig.at[pl.ds(half, C)], gsem)
            pltpu.async_copy(ptab.at[idxp_all.at[pl.ds(c * C, C)]],
                             pbig.at[pl.ds(half, C)], gsem)

        def wait_gather():
            pltpu.make_async_copy(wtab.at[pl.ds(0, C)],
                                  wbig.at[pl.ds(0, C)], gsem).wait()
            pltpu.make_async_copy(ptab.at[pl.ds(0, C)],
                                  pbig.at[pl.ds(0, C)], gsem).wait()

        def wait_store():
            pltpu.make_async_copy(wbig.at[pl.ds(0, C)],
                                  out_hbm.at[pl.ds(0, C)], ssem).wait()

        fire_gather(0)

        def chunk_body(c, carry):
            boff = (c & 1) * C
            toff = c * C
            wait_gather()
            pl.when(c >= 1)(wait_store)
            pl.when(c + 1 < n_chunks)(lambda: fire_gather(c + 1))

            for g in range(C // 16):
                ttv = idxt_all[pl.ds(toff + g * 16, 16)]
                for u in range(16):
                    tsm[g * 16 + u] = ttv[u]

            AI = 4

            def tok_body(tp, tc):
                ts = [tp * AI + u for u in range(AI)]
                rs = [boff + t for t in ts]
                tts = [tsm[t] for t in ts]
                zero = jnp.zeros((16,), jnp.float32)
                accs = [zero] * AI
                acc2s = [zero] * AI
                for j in range(NSLICE):
                    sl = pl.ds(j * 16, 16)
                    for u in range(AI):
                        x = wbig[rs[u], sl] + pbig[rs[u], sl] + ttab[tts[u], sl]
                        wbig[rs[u], sl] = x
                        accs[u] = accs[u] + x
                        acc2s[u] = acc2s[u] + x * x
                means = [_lane_sum16(a) * INV_H for a in accs]
                rstds = [_rsqrt16(_lane_sum16(a2) * INV_H - m * m + EPS)
                         for a2, m in zip(acc2s, means)]
                for u in range(AI):
                    mbuf[ts[u]] = means[u][0]
                    rbuf[ts[u]] = rstds[u][0]
                return tc

            lax.fori_loop(0, C // AI, tok_body, 0)

            for j in range(NSLICE):
                sl = pl.ds(j * 16, 16)
                g = gbuf[sl]
                bt = bbuf[sl]

                def norm_body(t8, tc2, sl=sl, g=g, bt=bt):
                    for u in range(8):
                        t = t8 * 8 + u
                        r = boff + t
                        x = wbig[r, sl]
                        wbig[r, sl] = (x - mbuf[t]) * rbuf[t] * g + bt
                    return tc2

                lax.fori_loop(0, C // 8, norm_body, 0)

            pltpu.async_copy(wbig.at[pl.ds(boff, C)],
                             out_hbm.at[pl.ds(base + toff, C)], ssem)
            return carry

        lax.fori_loop(0, n_chunks, chunk_body, 0)
        wait_store()

    return k(iw, ip, it, word_emb, pos_emb, tok_type_emb, gamma, beta)


def kernel(input_ids, token_type_ids, position_ids, word_emb, pos_emb, tok_type_emb, ln_gamma, ln_beta):
    B, S = input_ids.shape
    n = B * S
    iw = input_ids.reshape(n).astype(jnp.int32)
    it = token_type_ids.reshape(n).astype(jnp.int32)
    ip = position_ids.reshape(n).astype(jnp.int32)
    out = _sc_embed_ln(iw, ip, it, word_emb, pos_emb, tok_type_emb,
                       ln_gamma, ln_beta, n_tokens=n)
    return out.reshape(B, S, HIDDEN)


# speedup vs baseline: 1.1579x; 1.1579x over previous
import functools

import jax
import jax.numpy as jnp
from jax import lax
from jax.experimental import pallas as pl
from jax.experimental.pallas import tpu as pltpu
from jax.experimental.pallas import tpu_sc as plsc

HIDDEN = 768
NSLICE = HIDDEN // 16
EPS = 1e-5
INV_H = 1.0 / HIDDEN


def _lane_sum16(x):
    dnums = lax.GatherDimensionNumbers(
        offset_dims=(), collapsed_slice_dims=(0,), start_index_map=(0,))
    lane = lax.iota(jnp.int32, 16)
    for sh in (8, 4, 2, 1):
        perm = ((lane + sh) & 15).reshape(16, 1)
        x = x + lax.gather(x, perm, dnums, slice_sizes=(1,),
                           mode=lax.GatherScatterMode.PROMISE_IN_BOUNDS)
    return x


def _rsqrt16(x):
    i = plsc.bitcast(x, jnp.int32)
    i = jnp.int32(0x5F3759DF) - (i >> 1)
    y = plsc.bitcast(i, jnp.float32)
    for _ in range(3):
        y = y * (1.5 - 0.5 * x * y * y)
    return y


@functools.partial(jax.jit, static_argnames=("n_tokens",))
def _sc_embed_ln(iw, ip, it, word_emb, pos_emb, tok_type_emb, gamma, beta, *, n_tokens):
    info = plsc.get_sparse_core_info()
    nc, ns = info.num_cores, info.num_subcores
    nw = nc * ns
    t_per_w = n_tokens // nw
    C = 32
    n_chunks = t_per_w // C

    mesh = plsc.VectorSubcoreMesh(core_axis_name="c", subcore_axis_name="s")

    @functools.partial(
        pl.kernel,
        mesh=mesh,
        compiler_params=pltpu.CompilerParams(needs_layout_passes=False),
        out_type=jax.ShapeDtypeStruct((n_tokens, HIDDEN), jnp.float32),
        scratch_types=[
            pltpu.VMEM((t_per_w,), jnp.int32),
            pltpu.VMEM((t_per_w,), jnp.int32),
            pltpu.VMEM((t_per_w,), jnp.int32),
            pltpu.VMEM((2 * C, HIDDEN), jnp.float32),
            pltpu.VMEM((2 * C, HIDDEN), jnp.float32),
            pltpu.VMEM((2, HIDDEN), jnp.float32),
            pltpu.VMEM((HIDDEN,), jnp.float32),
            pltpu.VMEM((HIDDEN,), jnp.float32),
            pltpu.SMEM((C,), jnp.float32),
            pltpu.SMEM((C,), jnp.float32),
            pltpu.SMEM((C,), jnp.int32),
            pltpu.SemaphoreType.DMA,
            pltpu.SemaphoreType.DMA,
        ],
    )
    def k(iw_hbm, ip_hbm, it_hbm, wtab, ptab, ttab_hbm, g_hbm, b_hbm, out_hbm,
          idxw_all, idxp_all, idxt_all, wbig, pbig,
          ttab, gbuf, bbuf, sbuf, rbuf, tsm, gsem, ssem):
        wid = lax.axis_index("s") * nc + lax.axis_index("c")
        base = wid * t_per_w
        pltpu.sync_copy(iw_hbm.at[pl.ds(base, t_per_w)], idxw_all)
        pltpu.sync_copy(ip_hbm.at[pl.ds(base, t_per_w)], idxp_all)
        pltpu.sync_copy(it_hbm.at[pl.ds(base, t_per_w)], idxt_all)
        pltpu.sync_copy(ttab_hbm, ttab)
        pltpu.sync_copy(g_hbm, gbuf)
        pltpu.sync_copy(b_hbm, bbuf)

        def fire_gather(c):
            half = (c & 1) * C
            pltpu.async_copy(wtab.at[idxw_all.at[pl.ds(c * C, C)]],
                             wbig.at[pl.ds(half, C)], gsem)
            pltpu.async_copy(ptab.at[idxp_all.at[pl.ds(c * C, C)]],
                             pbig.at[pl.ds(half, C)], gsem)

        def wait_gather():
            pltpu.make_async_copy(wtab.at[pl.ds(0, C)],
                                  wbig.at[pl.ds(0, C)], gsem).wait()
            pltpu.make_async_copy(ptab.at[pl.ds(0, C)],
                                  pbig.at[pl.ds(0, C)], gsem).wait()

        def wait_store():
            pltpu.make_async_copy(wbig.at[pl.ds(0, C)],
                                  out_hbm.at[pl.ds(0, C)], ssem).wait()

        fire_gather(0)

        def chunk_body(c, carry):
            boff = (c & 1) * C
            toff = c * C
            wait_gather()
            pl.when(c >= 1)(wait_store)
            pl.when(c + 1 < n_chunks)(lambda: fire_gather(c + 1))

            for g in range(C // 16):
                ttv = idxt_all[pl.ds(toff + g * 16, 16)]
                for u in range(16):
                    tsm[g * 16 + u] = ttv[u]

            AI = 2

            def tok_body(tp, tc):
                ts = [tp * AI + u for u in range(AI)]
                rs = [boff + t for t in ts]
                tts = [tsm[t] for t in ts]
                zero = jnp.zeros((16,), jnp.float32)
                accs = [zero] * AI
                acc2s = [zero] * AI
                for j in range(NSLICE):
                    sl = pl.ds(j * 16, 16)
                    for u in range(AI):
                        x = wbig[rs[u], sl] + pbig[rs[u], sl] + ttab[tts[u], sl]
                        wbig[rs[u], sl] = x
                        accs[u] = accs[u] + x
                        acc2s[u] = acc2s[u] + x * x
                means = [_lane_sum16(a) * INV_H for a in accs]
                rstds = [_rsqrt16(_lane_sum16(a2) * INV_H - m * m + EPS)
                         for a2, m in zip(acc2s, means)]
                for u in range(AI):
                    sbuf[ts[u]] = (-means[u] * rstds[u])[0]
                    rbuf[ts[u]] = rstds[u][0]
                return tc

            lax.fori_loop(0, C // AI, tok_body, 0)

            for j in range(NSLICE):
                sl = pl.ds(j * 16, 16)
                g = gbuf[sl]
                bt = bbuf[sl]

                def norm_body(t16, tc2, sl=sl, g=g, bt=bt):
                    for u in range(16):
                        t = t16 * 16 + u
                        r = boff + t
                        x = wbig[r, sl]
                        wbig[r, sl] = (x * rbuf[t] + sbuf[t]) * g + bt
                    return tc2

                lax.fori_loop(0, C // 16, norm_body, 0)

            pltpu.async_copy(wbig.at[pl.ds(boff, C)],
                             out_hbm.at[pl.ds(base + toff, C)], ssem)
            return carry

        lax.fori_loop(0, n_chunks, chunk_body, 0)
        wait_store()

    return k(iw, ip, it, word_emb, pos_emb, tok_type_emb, gamma, beta)


def kernel(input_ids, token_type_ids, position_ids, word_emb, pos_emb, tok_type_emb, ln_gamma, ln_beta):
    B, S = input_ids.shape
    n = B * S
    iw = input_ids.reshape(n).astype(jnp.int32)
    it = token_type_ids.reshape(n).astype(jnp.int32)
    ip = position_ids.reshape(n).astype(jnp.int32)
    out = _sc_embed_ln(iw, ip, it, word_emb, pos_emb, tok_type_emb,
                       ln_gamma, ln_beta, n_tokens=n)
    return out.reshape(B, S, HIDDEN)


# speedup vs baseline: 1.6706x; 1.4428x over previous
import functools

import jax
import jax.numpy as jnp
from jax import lax
from jax.experimental import pallas as pl
from jax.experimental.pallas import tpu as pltpu
from jax.experimental.pallas import tpu_sc as plsc

HIDDEN = 768
NSLICE = HIDDEN // 16
EPS = 1e-5
INV_H = 1.0 / HIDDEN


def _lane_sum16(x):
    dnums = lax.GatherDimensionNumbers(
        offset_dims=(), collapsed_slice_dims=(0,), start_index_map=(0,))
    lane = lax.iota(jnp.int32, 16)
    for sh in (8, 4, 2, 1):
        perm = ((lane + sh) & 15).reshape(16, 1)
        x = x + lax.gather(x, perm, dnums, slice_sizes=(1,),
                           mode=lax.GatherScatterMode.PROMISE_IN_BOUNDS)
    return x


def _rsqrt16(x):
    i = plsc.bitcast(x, jnp.int32)
    i = jnp.int32(0x5F3759DF) - (i >> 1)
    y = plsc.bitcast(i, jnp.float32)
    for _ in range(3):
        y = y * (1.5 - 0.5 * x * y * y)
    return y


@functools.partial(jax.jit, static_argnames=("n_tokens",))
def _sc_embed_ln(iw, ip, it, word_emb, pos_emb, tok_type_emb, gamma, beta, *, n_tokens):
    info = plsc.get_sparse_core_info()
    nc, ns = info.num_cores, info.num_subcores
    nw = nc * ns
    t_per_w = n_tokens // nw
    C = 32
    n_chunks = t_per_w // C

    mesh = plsc.VectorSubcoreMesh(core_axis_name="c", subcore_axis_name="s")

    @functools.partial(
        pl.kernel,
        mesh=mesh,
        compiler_params=pltpu.CompilerParams(needs_layout_passes=False),
        out_type=jax.ShapeDtypeStruct((n_tokens, HIDDEN), jnp.float32),
        scratch_types=[
            pltpu.VMEM((t_per_w,), jnp.int32),
            pltpu.VMEM((t_per_w,), jnp.int32),
            pltpu.VMEM((t_per_w,), jnp.int32),
            pltpu.VMEM((2 * C, HIDDEN), jnp.float32),
            pltpu.VMEM((2 * C, HIDDEN), jnp.float32),
            pltpu.VMEM((2, HIDDEN), jnp.float32),
            pltpu.VMEM((HIDDEN,), jnp.float32),
            pltpu.VMEM((HIDDEN,), jnp.float32),
            pltpu.SMEM((C,), jnp.float32),
            pltpu.SMEM((C,), jnp.float32),
            pltpu.SMEM((C,), jnp.int32),
            pltpu.SemaphoreType.DMA,
            pltpu.SemaphoreType.DMA,
        ],
    )
    def k(iw_hbm, ip_hbm, it_hbm, wtab, ptab, ttab_hbm, g_hbm, b_hbm, out_hbm,
          idxw_all, idxp_all, idxt_all, wbig, pbig,
          ttab, gbuf, bbuf, sbuf, rbuf, tsm, gsem, ssem):
        wid = lax.axis_index("s") * nc + lax.axis_index("c")
        base = wid * t_per_w
        pltpu.sync_copy(iw_hbm.at[pl.ds(base, t_per_w)], idxw_all)
        pltpu.sync_copy(ip_hbm.at[pl.ds(base, t_per_w)], idxp_all)
        pltpu.sync_copy(it_hbm.at[pl.ds(base, t_per_w)], idxt_all)
        pltpu.sync_copy(ttab_hbm, ttab)
        pltpu.sync_copy(g_hbm, gbuf)
        pltpu.sync_copy(b_hbm, bbuf)

        def fire_gather(c):
            half = (c & 1) * C
            pltpu.async_copy(wtab.at[idxw_all.at[pl.ds(c * C, C)]],
                             wbig.at[pl.ds(half, C)], gsem)
            pltpu.async_copy(ptab.at[idxp_all.at[pl.ds(c * C, C)]],
                             pbig.at[pl.ds(half, C)], gsem)

        def wait_gather():
            pltpu.make_async_copy(wtab.at[pl.ds(0, C)],
                                  wbig.at[pl.ds(0, C)], gsem).wait()
            pltpu.make_async_copy(ptab.at[pl.ds(0, C)],
                                  pbig.at[pl.ds(0, C)], gsem).wait()

        def wait_store():
            pltpu.make_async_copy(wbig.at[pl.ds(0, C)],
                                  out_hbm.at[pl.ds(0, C)], ssem).wait()

        fire_gather(0)

        def chunk_body(c, carry):
            boff = (c & 1) * C
            toff = c * C
            wait_gather()
            pl.when(c >= 1)(wait_store)
            pl.when(c + 1 < n_chunks)(lambda: fire_gather(c + 1))

            for g in range(C // 16):
                ttv = idxt_all[pl.ds(toff + g * 16, 16)]
                for u in range(16):
                    tsm[g * 16 + u] = ttv[u]

            AI = 2

            def tok_body(tp, tc):
                ts = [tp * AI + u for u in range(AI)]
                rs = [boff + t for t in ts]
                tts = [tsm[t] for t in ts]
                zero = jnp.zeros((16,), jnp.float32)
                accs = [zero] * AI
                acc2s = [zero] * AI
                for j in range(NSLICE):
                    sl = pl.ds(j * 16, 16)
                    for u in range(AI):
                        x = wbig[rs[u], sl] + pbig[rs[u], sl] + ttab[tts[u], sl]
                        wbig[rs[u], sl] = x
                        accs[u] = accs[u] + x
                        acc2s[u] = acc2s[u] + x * x
                means = [_lane_sum16(a) * INV_H for a in accs]
                rstds = [_rsqrt16(_lane_sum16(a2) * INV_H - m * m + EPS)
                         for a2, m in zip(acc2s, means)]
                for u in range(AI):
                    sbuf[ts[u]] = (-means[u] * rstds[u])[0]
                    rbuf[ts[u]] = rstds[u][0]
                return tc

            lax.fori_loop(0, C // AI, tok_body, 0)

            for j in range(NSLICE):
                sl = pl.ds(j * 16, 16)
                g = gbuf[sl]
                bt = bbuf[sl]

                def norm_body(t8, tc2, sl=sl, g=g, bt=bt):
                    for u in range(8):
                        t = t8 * 8 + u
                        r = boff + t
                        x = wbig[r, sl]
                        wbig[r, sl] = (x * rbuf[t] + sbuf[t]) * g + bt
                    return tc2

                lax.fori_loop(0, C // 8, norm_body, 0)

            pltpu.async_copy(wbig.at[pl.ds(boff, C)],
                             out_hbm.at[pl.ds(base + toff, C)], ssem)
            return carry

        lax.fori_loop(0, n_chunks, chunk_body, 0)
        wait_store()

    return k(iw, ip, it, word_emb, pos_emb, tok_type_emb, gamma, beta)


def kernel(input_ids, token_type_ids, position_ids, word_emb, pos_emb, tok_type_emb, ln_gamma, ln_beta):
    B, S = input_ids.shape
    n = B * S
    iw = input_ids.reshape(n).astype(jnp.int32)
    it = token_type_ids.reshape(n).astype(jnp.int32)
    ip = position_ids.reshape(n).astype(jnp.int32)
    out = _sc_embed_ln(iw, ip, it, word_emb, pos_emb, tok_type_emb,
                       ln_gamma, ln_beta, n_tokens=n)
    return out.reshape(B, S, HIDDEN)


# speedup vs baseline: 1.6713x; 1.0004x over previous
import functools

import jax
import jax.numpy as jnp
from jax import lax
from jax.experimental import pallas as pl
from jax.experimental.pallas import tpu as pltpu
from jax.experimental.pallas import tpu_sc as plsc

HIDDEN = 768
NSLICE = HIDDEN // 16
EPS = 1e-5
INV_H = 1.0 / HIDDEN


def _lane_sum16(x):
    dnums = lax.GatherDimensionNumbers(
        offset_dims=(), collapsed_slice_dims=(0,), start_index_map=(0,))
    lane = lax.iota(jnp.int32, 16)
    for sh in (8, 4, 2, 1):
        perm = ((lane + sh) & 15).reshape(16, 1)
        x = x + lax.gather(x, perm, dnums, slice_sizes=(1,),
                           mode=lax.GatherScatterMode.PROMISE_IN_BOUNDS)
    return x


def _rsqrt16(x):
    i = plsc.bitcast(x, jnp.int32)
    i = jnp.int32(0x5F3759DF) - (i >> 1)
    y = plsc.bitcast(i, jnp.float32)
    for _ in range(3):
        y = y * (1.5 - 0.5 * x * y * y)
    return y


@functools.partial(jax.jit, static_argnames=("n_tokens",))
def _sc_embed_ln(iw, ip, it, word_emb, pos_emb, tok_type_emb, gamma, beta, *, n_tokens):
    info = plsc.get_sparse_core_info()
    nc, ns = info.num_cores, info.num_subcores
    nw = nc * ns
    t_per_w = n_tokens // nw
    C = 32
    n_chunks = t_per_w // C

    mesh = plsc.VectorSubcoreMesh(core_axis_name="c", subcore_axis_name="s")

    @functools.partial(
        pl.kernel,
        mesh=mesh,
        compiler_params=pltpu.CompilerParams(needs_layout_passes=False),
        out_type=jax.ShapeDtypeStruct((n_tokens, HIDDEN), jnp.float32),
        scratch_types=[
            pltpu.VMEM((t_per_w,), jnp.int32),
            pltpu.VMEM((t_per_w,), jnp.int32),
            pltpu.VMEM((t_per_w,), jnp.int32),
            pltpu.VMEM((2 * C, HIDDEN), jnp.float32),
            pltpu.VMEM((2 * C, HIDDEN), jnp.float32),
            pltpu.VMEM((2, HIDDEN), jnp.float32),
            pltpu.VMEM((HIDDEN,), jnp.float32),
            pltpu.VMEM((HIDDEN,), jnp.float32),
            pltpu.SMEM((C,), jnp.float32),
            pltpu.SMEM((C,), jnp.float32),
            pltpu.SMEM((C,), jnp.int32),
            pltpu.SemaphoreType.DMA,
            pltpu.SemaphoreType.DMA,
        ],
    )
    def k(iw_hbm, ip_hbm, it_hbm, wtab, ptab, ttab_hbm, g_hbm, b_hbm, out_hbm,
          idxw_all, idxp_all, idxt_all, wbig, pbig,
          ttab, gbuf, bbuf, sbuf, rbuf, tsm, gsem, ssem):
        wid = lax.axis_index("s") * nc + lax.axis_index("c")
        base = wid * t_per_w
        pltpu.sync_copy(iw_hbm.at[pl.ds(base, t_per_w)], idxw_all)
        pltpu.sync_copy(ip_hbm.at[pl.ds(base, t_per_w)], idxp_all)
        pltpu.sync_copy(it_hbm.at[pl.ds(base, t_per_w)], idxt_all)
        pltpu.sync_copy(ttab_hbm, ttab)
        pltpu.sync_copy(g_hbm, gbuf)
        pltpu.sync_copy(b_hbm, bbuf)

        def fire_gather(c):
            half = (c & 1) * C
            pltpu.async_copy(wtab.at[idxw_all.at[pl.ds(c * C, C)]],
                             wbig.at[pl.ds(half, C)], gsem)
            pltpu.async_copy(ptab.at[idxp_all.at[pl.ds(c * C, C)]],
                             pbig.at[pl.ds(half, C)], gsem)

        def wait_gather():
            pltpu.make_async_copy(wtab.at[pl.ds(0, C)],
                                  wbig.at[pl.ds(0, C)], gsem).wait()
            pltpu.make_async_copy(ptab.at[pl.ds(0, C)],
                                  pbig.at[pl.ds(0, C)], gsem).wait()

        def wait_store():
            pltpu.make_async_copy(wbig.at[pl.ds(0, C)],
                                  out_hbm.at[pl.ds(0, C)], ssem).wait()

        fire_gather(0)

        def chunk_body(c, carry):
            boff = (c & 1) * C
            toff = c * C
            wait_gather()
            pl.when(c >= 1)(wait_store)
            pl.when(c + 1 < n_chunks)(lambda: fire_gather(c + 1))

            for g in range(C // 16):
                ttv = idxt_all[pl.ds(toff + g * 16, 16)]
                for u in range(16):
                    tsm[g * 16 + u] = ttv[u]

            AI = 2

            def tok_body(tp, tc):
                ts = [tp * AI + u for u in range(AI)]
                rs = [boff + t for t in ts]
                tts = [tsm[t] for t in ts]
                zero = jnp.zeros((16,), jnp.float32)
                accs = [zero] * AI
                acc2s = [zero] * AI
                for j in range(NSLICE):
                    sl = pl.ds(j * 16, 16)
                    for u in range(AI):
                        x = wbig[rs[u], sl] + pbig[rs[u], sl] + ttab[tts[u], sl]
                        wbig[rs[u], sl] = x
                        accs[u] = accs[u] + x
                        acc2s[u] = acc2s[u] + x * x
                means = [_lane_sum16(a) * INV_H for a in accs]
                rstds = [_rsqrt16(_lane_sum16(a2) * INV_H - m * m + EPS)
                         for a2, m in zip(acc2s, means)]
                for u in range(AI):
                    sbuf[ts[u]] = means[u][0]
                    rbuf[ts[u]] = rstds[u][0]
                return tc

            lax.fori_loop(0, C // AI, tok_body, 0)

            for j in range(NSLICE):
                sl = pl.ds(j * 16, 16)
                g = gbuf[sl]
                bt = bbuf[sl]

                def norm_body(t8, tc2, sl=sl, g=g, bt=bt):
                    for u in range(8):
                        t = t8 * 8 + u
                        r = boff + t
                        x = wbig[r, sl]
                        wbig[r, sl] = (x - sbuf[t]) * rbuf[t] * g + bt
                    return tc2

                lax.fori_loop(0, C // 8, norm_body, 0)

            pltpu.async_copy(wbig.at[pl.ds(boff, C)],
                             out_hbm.at[pl.ds(base + toff, C)], ssem)
            return carry

        lax.fori_loop(0, n_chunks, chunk_body, 0)
        wait_store()

    return k(iw, ip, it, word_emb, pos_emb, tok_type_emb, gamma, beta)


def kernel(input_ids, token_type_ids, position_ids, word_emb, pos_emb, tok_type_emb, ln_gamma, ln_beta):
    B, S = input_ids.shape
    n = B * S
    iw = input_ids.reshape(n).astype(jnp.int32)
    it = token_type_ids.reshape(n).astype(jnp.int32)
    ip = position_ids.reshape(n).astype(jnp.int32)
    out = _sc_embed_ln(iw, ip, it, word_emb, pos_emb, tok_type_emb,
                       ln_gamma, ln_beta, n_tokens=n)
    return out.reshape(B, S, HIDDEN)


# speedup vs baseline: 2.2875x; 1.3687x over previous
import functools

import jax
import jax.numpy as jnp
from jax import lax
from jax.experimental import pallas as pl
from jax.experimental.pallas import tpu as pltpu
from jax.experimental.pallas import tpu_sc as plsc

HIDDEN = 768
NSLICE = HIDDEN // 16
EPS = 1e-5
INV_H = 1.0 / HIDDEN


def _lane_sum16(x):
    dnums = lax.GatherDimensionNumbers(
        offset_dims=(), collapsed_slice_dims=(0,), start_index_map=(0,))
    lane = lax.iota(jnp.int32, 16)
    for sh in (8, 4, 2, 1):
        perm = ((lane + sh) & 15).reshape(16, 1)
        x = x + lax.gather(x, perm, dnums, slice_sizes=(1,),
                           mode=lax.GatherScatterMode.PROMISE_IN_BOUNDS)
    return x


def _rsqrt16(x):
    i = plsc.bitcast(x, jnp.int32)
    i = jnp.int32(0x5F3759DF) - (i >> 1)
    y = plsc.bitcast(i, jnp.float32)
    for _ in range(3):
        y = y * (1.5 - 0.5 * x * y * y)
    return y


@functools.partial(jax.jit, static_argnames=("n_tokens",))
def _sc_embed_ln(iw, ip, it, word_emb, pos_emb, tok_type_emb, gamma, beta, *, n_tokens):
    info = plsc.get_sparse_core_info()
    nc, ns = info.num_cores, info.num_subcores
    nw = nc * ns
    t_per_w = n_tokens // nw
    C = 32
    n_chunks = t_per_w // C

    mesh = plsc.VectorSubcoreMesh(core_axis_name="c", subcore_axis_name="s")

    @functools.partial(
        pl.kernel,
        mesh=mesh,
        compiler_params=pltpu.CompilerParams(needs_layout_passes=False),
        out_type=jax.ShapeDtypeStruct((n_tokens, HIDDEN), jnp.float32),
        scratch_types=[
            pltpu.VMEM((t_per_w,), jnp.int32),
            pltpu.VMEM((t_per_w,), jnp.int32),
            pltpu.VMEM((t_per_w,), jnp.int32),
            pltpu.VMEM((2 * C, HIDDEN), jnp.float32),
            pltpu.VMEM((2 * C, HIDDEN), jnp.float32),
            pltpu.VMEM((2, HIDDEN), jnp.float32),
            pltpu.VMEM((HIDDEN,), jnp.float32),
            pltpu.VMEM((HIDDEN,), jnp.float32),
            pltpu.SMEM((C,), jnp.float32),
            pltpu.SMEM((C,), jnp.float32),
            pltpu.SMEM((C,), jnp.int32),
            pltpu.SemaphoreType.DMA,
            pltpu.SemaphoreType.DMA,
        ],
    )
    def k(iw_hbm, ip_hbm, it_hbm, wtab, ptab, ttab_hbm, g_hbm, b_hbm, out_hbm,
          idxw_all, idxp_all, idxt_all, wbig, pbig,
          ttab, gbuf, bbuf, sbuf, rbuf, tsm, gsem, ssem):
        wid = lax.axis_index("s") * nc + lax.axis_index("c")
        base = wid * t_per_w
        pltpu.sync_copy(iw_hbm.at[pl.ds(base, t_per_w)], idxw_all)
        pltpu.sync_copy(ip_hbm.at[pl.ds(base, t_per_w)], idxp_all)
        pltpu.sync_copy(it_hbm.at[pl.ds(base, t_per_w)], idxt_all)
        pltpu.sync_copy(ttab_hbm, ttab)
        pltpu.sync_copy(g_hbm, gbuf)
        pltpu.sync_copy(b_hbm, bbuf)

        def fire_gather(c):
            half = (c & 1) * C
            pltpu.async_copy(wtab.at[idxw_all.at[pl.ds(c * C, C)]],
                             wbig.at[pl.ds(half, C)], gsem)
            pltpu.async_copy(ptab.at[idxp_all.at[pl.ds(c * C, C)]],
                             pbig.at[pl.ds(half, C)], gsem)

        def wait_gather():
            pltpu.make_async_copy(wtab.at[pl.ds(0, C)],
                                  wbig.at[pl.ds(0, C)], gsem).wait()
            pltpu.make_async_copy(ptab.at[pl.ds(0, C)],
                                  pbig.at[pl.ds(0, C)], gsem).wait()

        def wait_store():
            pltpu.make_async_copy(wbig.at[pl.ds(0, C)],
                                  out_hbm.at[pl.ds(0, C)], ssem).wait()

        fire_gather(0)

        def chunk_body(c, carry):
            boff = (c & 1) * C
            toff = c * C
            wait_gather()
            pl.when(c >= 1)(wait_store)
            pl.when(c + 1 < n_chunks)(lambda: fire_gather(c + 1))

            for g in range(C // 16):
                ttv = idxt_all[pl.ds(toff + g * 16, 16)]
                for u in range(16):
                    tsm[g * 16 + u] = ttv[u]

            def tok_body(tp, tc):
                t0 = tp * 2
                t1 = t0 + 1
                r0 = boff + t0
                r1 = boff + t1
                tt0 = tsm[t0]
                tt1 = tsm[t1]
                acc0 = jnp.zeros((16,), jnp.float32)
                acc2_0 = jnp.zeros((16,), jnp.float32)
                acc1 = jnp.zeros((16,), jnp.float32)
                acc2_1 = jnp.zeros((16,), jnp.float32)
                for j in range(NSLICE):
                    sl = pl.ds(j * 16, 16)
                    x0 = wbig[r0, sl] + pbig[r0, sl] + ttab[tt0, sl]
                    x1 = wbig[r1, sl] + pbig[r1, sl] + ttab[tt1, sl]
                    wbig[r0, sl] = x0
                    wbig[r1, sl] = x1
                    acc0 = acc0 + x0
                    acc2_0 = acc2_0 + x0 * x0
                    acc1 = acc1 + x1
                    acc2_1 = acc2_1 + x1 * x1
                mean0 = _lane_sum16(acc0) * INV_H
                mean1 = _lane_sum16(acc1) * INV_H
                var0 = _lane_sum16(acc2_0) * INV_H - mean0 * mean0
                var1 = _lane_sum16(acc2_1) * INV_H - mean1 * mean1
                rstd0 = _rsqrt16(var0 + EPS)
                rstd1 = _rsqrt16(var1 + EPS)
                sbuf[t0] = mean0[0]
                sbuf[t1] = mean1[0]
                rbuf[t0] = rstd0[0]
                rbuf[t1] = rstd1[0]
                return tc

            lax.fori_loop(0, C // 2, tok_body, 0)

            for j in range(NSLICE):
                sl = pl.ds(j * 16, 16)
                g = gbuf[sl]
                bt = bbuf[sl]

                def norm_body(t8, tc2, sl=sl, g=g, bt=bt):
                    for u in range(8):
                        t = t8 * 8 + u
                        r = boff + t
                        x = wbig[r, sl]
                        wbig[r, sl] = (x - sbuf[t]) * rbuf[t] * g + bt
                    return tc2

                lax.fori_loop(0, C // 8, norm_body, 0)

            pltpu.async_copy(wbig.at[pl.ds(boff, C)],
                             out_hbm.at[pl.ds(base + toff, C)], ssem)
            return carry

        lax.fori_loop(0, n_chunks, chunk_body, 0)
        wait_store()

    return k(iw, ip, it, word_emb, pos_emb, tok_type_emb, gamma, beta)


def kernel(input_ids, token_type_ids, position_ids, word_emb, pos_emb, tok_type_emb, ln_gamma, ln_beta):
    B, S = input_ids.shape
    n = B * S
    iw = input_ids.reshape(n).astype(jnp.int32)
    it = token_type_ids.reshape(n).astype(jnp.int32)
    ip = position_ids.reshape(n).astype(jnp.int32)
    out = _sc_embed_ln(iw, ip, it, word_emb, pos_emb, tok_type_emb,
                       ln_gamma, ln_beta, n_tokens=n)
    return out.reshape(B, S, HIDDEN)


# speedup vs baseline: 2.2909x; 1.0015x over previous
import functools

import jax
import jax.numpy as jnp
from jax import lax
from jax.experimental import pallas as pl
from jax.experimental.pallas import tpu as pltpu
from jax.experimental.pallas import tpu_sc as plsc

HIDDEN = 768
NSLICE = HIDDEN // 16
EPS = 1e-5
INV_H = 1.0 / HIDDEN


def _lane_sum16(x):
    dnums = lax.GatherDimensionNumbers(
        offset_dims=(), collapsed_slice_dims=(0,), start_index_map=(0,))
    lane = lax.iota(jnp.int32, 16)
    for sh in (8, 4, 2, 1):
        perm = ((lane + sh) & 15).reshape(16, 1)
        x = x + lax.gather(x, perm, dnums, slice_sizes=(1,),
                           mode=lax.GatherScatterMode.PROMISE_IN_BOUNDS)
    return x


def _rsqrt16(x):
    i = plsc.bitcast(x, jnp.int32)
    i = jnp.int32(0x5F3759DF) - (i >> 1)
    y = plsc.bitcast(i, jnp.float32)
    for _ in range(3):
        y = y * (1.5 - 0.5 * x * y * y)
    return y


@functools.partial(jax.jit, static_argnames=("n_tokens",))
def _sc_embed_ln(iw, ip, it, word_emb, pos_emb, tok_type_emb, gamma, beta, *, n_tokens):
    info = plsc.get_sparse_core_info()
    nc, ns = info.num_cores, info.num_subcores
    nw = nc * ns
    t_per_w = n_tokens // nw
    C = 32
    n_chunks = t_per_w // C

    mesh = plsc.VectorSubcoreMesh(core_axis_name="c", subcore_axis_name="s")

    @functools.partial(
        pl.kernel,
        mesh=mesh,
        compiler_params=pltpu.CompilerParams(needs_layout_passes=False),
        out_type=jax.ShapeDtypeStruct((n_tokens, HIDDEN), jnp.float32),
        scratch_types=[
            pltpu.VMEM((t_per_w,), jnp.int32),
            pltpu.VMEM((t_per_w,), jnp.int32),
            pltpu.VMEM((t_per_w,), jnp.int32),
            pltpu.VMEM((2 * C, HIDDEN), jnp.float32),
            pltpu.VMEM((2 * C, HIDDEN), jnp.float32),
            pltpu.VMEM((2, HIDDEN), jnp.float32),
            pltpu.VMEM((HIDDEN,), jnp.float32),
            pltpu.VMEM((HIDDEN,), jnp.float32),
            pltpu.SMEM((C,), jnp.float32),
            pltpu.SMEM((C,), jnp.float32),
            pltpu.SMEM((C,), jnp.int32),
            pltpu.SemaphoreType.DMA,
            pltpu.SemaphoreType.DMA,
        ],
    )
    def k(iw_hbm, ip_hbm, it_hbm, wtab, ptab, ttab_hbm, g_hbm, b_hbm, out_hbm,
          idxw_all, idxp_all, idxt_all, wbig, pbig,
          ttab, gbuf, bbuf, sbuf, rbuf, tsm, gsem, ssem):
        wid = lax.axis_index("s") * nc + lax.axis_index("c")
        base = wid * t_per_w
        pltpu.sync_copy(iw_hbm.at[pl.ds(base, t_per_w)], idxw_all)
        pltpu.sync_copy(ip_hbm.at[pl.ds(base, t_per_w)], idxp_all)
        pltpu.sync_copy(it_hbm.at[pl.ds(base, t_per_w)], idxt_all)
        pltpu.sync_copy(ttab_hbm, ttab)
        pltpu.sync_copy(g_hbm, gbuf)
        pltpu.sync_copy(b_hbm, bbuf)

        def fire_gather(c):
            half = (c & 1) * C
            pltpu.async_copy(wtab.at[idxw_all.at[pl.ds(c * C, C)]],
                             wbig.at[pl.ds(half, C)], gsem)
            pltpu.async_copy(ptab.at[idxp_all.at[pl.ds(c * C, C)]],
                             pbig.at[pl.ds(half, C)], gsem)

        def wait_gather():
            pltpu.make_async_copy(wtab.at[pl.ds(0, C)],
                                  wbig.at[pl.ds(0, C)], gsem).wait()
            pltpu.make_async_copy(ptab.at[pl.ds(0, C)],
                                  pbig.at[pl.ds(0, C)], gsem).wait()

        def wait_store():
            pltpu.make_async_copy(wbig.at[pl.ds(0, C)],
                                  out_hbm.at[pl.ds(0, C)], ssem).wait()

        fire_gather(0)

        def chunk_body(c, carry):
            boff = (c & 1) * C
            toff = c * C
            wait_gather()
            pl.when(c >= 1)(wait_store)
            pl.when(c + 1 < n_chunks)(lambda: fire_gather(c + 1))

            for g in range(C // 16):
                ttv = idxt_all[pl.ds(toff + g * 16, 16)]
                for u in range(16):
                    tsm[g * 16 + u] = ttv[u]

            def tok_body(tp, tc):
                t0 = tp * 2
                t1 = t0 + 1
                r0 = boff + t0
                r1 = boff + t1
                tt0 = tsm[t0]
                tt1 = tsm[t1]
                acc0 = jnp.zeros((16,), jnp.float32)
                acc2_0 = jnp.zeros((16,), jnp.float32)
                acc1 = jnp.zeros((16,), jnp.float32)
                acc2_1 = jnp.zeros((16,), jnp.float32)
                for j in range(NSLICE):
                    sl = pl.ds(j * 16, 16)
                    x0 = wbig[r0, sl] + pbig[r0, sl] + ttab[tt0, sl]
                    x1 = wbig[r1, sl] + pbig[r1, sl] + ttab[tt1, sl]
                    wbig[r0, sl] = x0
                    wbig[r1, sl] = x1
                    acc0 = acc0 + x0
                    acc2_0 = acc2_0 + x0 * x0
                    acc1 = acc1 + x1
                    acc2_1 = acc2_1 + x1 * x1
                mean0 = _lane_sum16(acc0) * INV_H
                mean1 = _lane_sum16(acc1) * INV_H
                var0 = _lane_sum16(acc2_0) * INV_H - mean0 * mean0
                var1 = _lane_sum16(acc2_1) * INV_H - mean1 * mean1
                rstd0 = _rsqrt16(var0 + EPS)
                rstd1 = _rsqrt16(var1 + EPS)
                sbuf[t0] = mean0[0]
                sbuf[t1] = mean1[0]
                rbuf[t0] = rstd0[0]
                rbuf[t1] = rstd1[0]
                return tc

            lax.fori_loop(0, C // 2, tok_body, 0)

            for j in range(NSLICE):
                sl = pl.ds(j * 16, 16)
                g = gbuf[sl]
                bt = bbuf[sl]

                def norm_body(t8, tc2, sl=sl, g=g, bt=bt):
                    ts = [t8 * 8 + u for u in range(8)]
                    xs = [wbig[boff + t, sl] for t in ts]
                    ys = [(x - sbuf[t]) * rbuf[t] for x, t in zip(xs, ts)]
                    for t, y in zip(ts, ys):
                        wbig[boff + t, sl] = y * g + bt
                    return tc2

                lax.fori_loop(0, C // 8, norm_body, 0)

            pltpu.async_copy(wbig.at[pl.ds(boff, C)],
                             out_hbm.at[pl.ds(base + toff, C)], ssem)
            return carry

        lax.fori_loop(0, n_chunks, chunk_body, 0)
        wait_store()

    return k(iw, ip, it, word_emb, pos_emb, tok_type_emb, gamma, beta)


def kernel(input_ids, token_type_ids, position_ids, word_emb, pos_emb, tok_type_emb, ln_gamma, ln_beta):
    B, S = input_ids.shape
    n = B * S
    iw = input_ids.reshape(n).astype(jnp.int32)
    it = token_type_ids.reshape(n).astype(jnp.int32)
    ip = position_ids.reshape(n).astype(jnp.int32)
    out = _sc_embed_ln(iw, ip, it, word_emb, pos_emb, tok_type_emb,
                       ln_gamma, ln_beta, n_tokens=n)
    return out.reshape(B, S, HIDDEN)
